# baseline (speedup 1.0000x reference)
import functools

import jax
import jax.numpy as jnp
from jax import lax
from jax.experimental import pallas as pl
from jax.experimental.pallas import tpu as pltpu
from jax.experimental.pallas import tpu_sc as plsc

BATCH = 16384
EMBED_DIM = 32
NUM_WORKERS = 32
B_PER_W = BATCH // NUM_WORKERS
CHUNK = 128
N_CHUNKS = B_PER_W // CHUNK
LANES = 16


def _body(uids_hbm, iids_hbm, utab_hbm, itab_hbm, out_hbm,
          uidx_v, iidx_v, urows_v, irows_v, out_v, sem):
    wid = lax.axis_index("s") * 2 + lax.axis_index("c")
    base = wid * B_PER_W

    pltpu.sync_copy(uids_hbm.at[pl.ds(wid * N_CHUNKS, N_CHUNKS)], uidx_v)
    pltpu.sync_copy(iids_hbm.at[pl.ds(wid * N_CHUNKS, N_CHUNKS)], iidx_v)

    copies = []
    for j in range(N_CHUNKS):
        copies.append(pltpu.async_copy(
            utab_hbm.at[uidx_v.at[j]],
            urows_v.at[pl.ds(j * CHUNK, CHUNK)], sem))
        copies.append(pltpu.async_copy(
            itab_hbm.at[iidx_v.at[j]],
            irows_v.at[pl.ds(j * CHUNK, CHUNK)], sem))
    for c in copies:
        c.wait()

    iota16 = lax.iota(jnp.int32, LANES)

    def group_body(g, carry):
        rows = g * LANES + iota16
        acc = jnp.zeros((LANES,), jnp.float32)
        for d in range(EMBED_DIM):
            dvec = jnp.full((LANES,), d, jnp.int32)
            uc = plsc.load_gather(urows_v, [rows, dvec])
            ic = plsc.load_gather(irows_v, [rows, dvec])
            acc = acc + uc * ic
        e = jnp.exp(-jnp.abs(acc))
        num = jnp.where(acc >= 0, jnp.ones_like(acc), e)
        plsc.store_scatter(out_v, [rows], num / (1.0 + e))
        return carry

    lax.fori_loop(0, B_PER_W // LANES, group_body, 0)

    pltpu.sync_copy(out_v, out_hbm.at[pl.ds(base, B_PER_W)])


@jax.jit
def kernel(user_ids, item_ids, user_table, item_table):
    uids = user_ids.astype(jnp.int32).reshape(NUM_WORKERS * N_CHUNKS, CHUNK)
    iids = item_ids.astype(jnp.int32).reshape(NUM_WORKERS * N_CHUNKS, CHUNK)

    mesh = plsc.VectorSubcoreMesh(core_axis_name="c", subcore_axis_name="s")
    run = pl.kernel(
        _body, mesh=mesh,
        out_type=jax.ShapeDtypeStruct((BATCH,), jnp.float32),
        compiler_params=pltpu.CompilerParams(
            use_tc_tiling_on_sc=False, needs_layout_passes=False),
        scratch_types=[
            pltpu.VMEM((N_CHUNKS, CHUNK), jnp.int32),
            pltpu.VMEM((N_CHUNKS, CHUNK), jnp.int32),
            pltpu.VMEM((B_PER_W, EMBED_DIM), jnp.float32),
            pltpu.VMEM((B_PER_W, EMBED_DIM), jnp.float32),
            pltpu.VMEM((B_PER_W,), jnp.float32),
            pltpu.SemaphoreType.DMA,
        ],
    )
    return run(uids, iids, user_table, item_table)
